# Initial kernel scaffold; baseline (speedup 1.0000x reference)
import jax
import jax.numpy as jnp
from jax.experimental import pallas as pl


def kernel(x, edge_attr, edge_index, cart_dist, Wg1, bg1, Wg2, bg2, Wa1, ba1, Wa2, ba2, gamma1, beta1, gamma2, beta2):
    raise NotImplementedError("write your pallas kernel here")


# speedup vs baseline: 2.3440x; 2.3440x over previous
import functools

import jax
import jax.numpy as jnp
from jax import lax
from jax.experimental import pallas as pl
from jax.experimental.pallas import tpu as pltpu
from jax.experimental.pallas import tpu_sc as plsc

N = 10000
E = 320000
D = 128
RADIUS = 5.0

NC = 2
NS = 16
NW = NC * NS

GBLK = 128
BE = 1600


def _silu(z):
    return z * jax.nn.sigmoid(z)


def _sc_gather(x, src, dst):
    nblk = E // GBLK
    per, rem = nblk // NW, nblk % NW
    mesh = plsc.VectorSubcoreMesh(core_axis_name="c", subcore_axis_name="s")

    @functools.partial(
        pl.kernel,
        out_type=(jax.ShapeDtypeStruct((E, D), x.dtype),
                  jax.ShapeDtypeStruct((E, D), x.dtype)),
        mesh=mesh,
        scratch_types=[
            pltpu.VMEM((GBLK,), jnp.int32),
            pltpu.VMEM((GBLK,), jnp.int32),
            pltpu.VMEM((GBLK, D), x.dtype),
            pltpu.VMEM((GBLK, D), x.dtype),
        ],
    )
    def k(x_hbm, src_hbm, dst_hbm, xi_hbm, xj_hbm, di_v, si_v, ri_v, rj_v):
        wid = lax.axis_index("s") * NC + lax.axis_index("c")
        start = wid * per + jnp.minimum(wid, rem)
        cnt = per + (wid < rem).astype(jnp.int32)

        @pl.loop(0, cnt)
        def _(t):
            b = (start + t) * GBLK
            pltpu.sync_copy(dst_hbm.at[pl.ds(b, GBLK)], di_v)
            pltpu.sync_copy(src_hbm.at[pl.ds(b, GBLK)], si_v)
            pltpu.sync_copy(x_hbm.at[di_v], ri_v)
            pltpu.sync_copy(x_hbm.at[si_v], rj_v)
            pltpu.sync_copy(ri_v, xi_hbm.at[pl.ds(b, GBLK)])
            pltpu.sync_copy(rj_v, xj_hbm.at[pl.ds(b, GBLK)])

    return k(x, src, dst)


def _p1_body(xi, xj, ea, wg1a, wg1b, wg1c, bg1, wg2, bg2, stats):
    i = pl.program_id(0)
    pre = (jnp.dot(xi[...], wg1a[...], preferred_element_type=jnp.float32)
           + jnp.dot(xj[...], wg1b[...], preferred_element_type=jnp.float32)
           + jnp.dot(ea[...], wg1c[...], preferred_element_type=jnp.float32)
           + bg1[...])
    u = jnp.dot(_silu(pre), wg2[...], preferred_element_type=jnp.float32) + bg2[...]
    su = jnp.sum(u, axis=0)
    ss = jnp.sum(u * u, axis=0)
    blk = jnp.concatenate([su[None], ss[None], jnp.zeros((6, D), jnp.float32)], axis=0)

    @pl.when(i == 0)
    def _():
        stats[...] = jnp.zeros_like(stats)

    stats[...] += blk


def _tc_pass1(xi, xj, ea, wg1a, wg1b, wg1c, bg1, wg2, bg2):
    wspec = pl.BlockSpec((D, D), lambda i: (0, 0))
    bspec = pl.BlockSpec((1, D), lambda i: (0, 0))
    return pl.pallas_call(
        _p1_body,
        grid=(E // BE,),
        in_specs=[
            pl.BlockSpec((BE, D), lambda i: (i, 0)),
            pl.BlockSpec((BE, D), lambda i: (i, 0)),
            pl.BlockSpec((BE, D), lambda i: (i, 0)),
            wspec, wspec, wspec, bspec, wspec, bspec,
        ],
        out_specs=pl.BlockSpec((8, D), lambda i: (0, 0)),
        out_shape=jax.ShapeDtypeStruct((8, D), jnp.float32),
    )(xi, xj, ea, wg1a, wg1b, wg1c, bg1, wg2, bg2)


def _p2_body(xi, xj, ea, cd, stats, wg1a, wg1b, wg1c, bg1, wg2, bg2,
             wa1a, wa1b, wa1c, ba1, wa2, ba2, g1, b1, eo, msg):
    pre_g = (jnp.dot(xi[...], wg1a[...], preferred_element_type=jnp.float32)
             + jnp.dot(xj[...], wg1b[...], preferred_element_type=jnp.float32)
             + jnp.dot(ea[...], wg1c[...], preferred_element_type=jnp.float32)
             + bg1[...])
    u = jnp.dot(_silu(pre_g), wg2[...], preferred_element_type=jnp.float32) + bg2[...]

    m = stats[0, :] * (1.0 / E)
    v = stats[1, :] * (1.0 / E) - m * m
    scale = g1[...] * jax.lax.rsqrt(v[None] + 1e-5)
    shift = b1[...] - m[None] * scale
    e_ij = jax.nn.sigmoid(u * scale + shift)

    d = cd[0, 0, :]
    env = 0.5 * (jnp.cos(d * (jnp.pi / RADIUS)) + 1.0) * (d < RADIUS).astype(jnp.float32)
    sigma = env[:, None] * e_ij

    pre_a = (jnp.dot(xi[...], wa1a[...], preferred_element_type=jnp.float32)
             + jnp.dot(xj[...], wa1b[...], preferred_element_type=jnp.float32)
             + jnp.dot(ea[...], wa1c[...], preferred_element_type=jnp.float32)
             + ba1[...])
    sender = jnp.dot(_silu(pre_a), wa2[...], preferred_element_type=jnp.float32) + ba2[...]

    eo[...] = ea[...] + sigma
    msg[...] = sigma * sender


def _tc_pass2(xi, xj, ea, cd3, stats, wg1a, wg1b, wg1c, bg1, wg2, bg2,
              wa1a, wa1b, wa1c, ba1, wa2, ba2, g1, b1):
    wspec = pl.BlockSpec((D, D), lambda i: (0, 0))
    bspec = pl.BlockSpec((1, D), lambda i: (0, 0))
    espec = pl.BlockSpec((BE, D), lambda i: (i, 0))
    return pl.pallas_call(
        _p2_body,
        grid=(E // BE,),
        in_specs=[
            espec, espec, espec,
            pl.BlockSpec((1, 1, BE), lambda i: (i, 0, 0)),
            pl.BlockSpec((8, D), lambda i: (0, 0)),
            wspec, wspec, wspec, bspec, wspec, bspec,
            wspec, wspec, wspec, bspec, wspec, bspec,
            bspec, bspec,
        ],
        out_specs=[espec, espec],
        out_shape=[jax.ShapeDtypeStruct((E, D), jnp.float32),
                   jax.ShapeDtypeStruct((E, D), jnp.float32)],
    )(xi, xj, ea, cd3, stats, wg1a, wg1b, wg1c, bg1, wg2, bg2,
      wa1a, wa1b, wa1c, ba1, wa2, ba2, g1, b1)


ZCH = 16
NCHUNK = N // ZCH


def _sc_scatter(msg, dst):
    nblk_core = (E // GBLK) // NC
    per, rem = nblk_core // NS, nblk_core % NS
    cper, crem = NCHUNK // NS, NCHUNK % NS
    mesh = plsc.VectorSubcoreMesh(core_axis_name="c", subcore_axis_name="s")

    @functools.partial(
        pl.kernel,
        out_type=jax.ShapeDtypeStruct((NC, N, D), jnp.float32),
        mesh=mesh,
        scratch_types=[
            pltpu.VMEM((GBLK,), jnp.int32),
            pltpu.VMEM((GBLK, D), jnp.float32),
            pltpu.VMEM((ZCH, D), jnp.float32),
            pltpu.VMEM_SHARED((N, D), jnp.float32),
        ],
    )
    def k(msg_hbm, dst_hbm, out_hbm, idx_v, val_v, zbuf, acc_sh):
        cid = lax.axis_index("c")
        sid = lax.axis_index("s")

        for r in range(ZCH):
            for c in range(D // 16):
                zbuf[r, pl.ds(c * 16, 16)] = jnp.zeros((16,), jnp.float32)
        cstart = sid * cper + jnp.minimum(sid, crem)
        ccnt = cper + (sid < crem).astype(jnp.int32)

        @pl.loop(0, ccnt)
        def _(t):
            pltpu.sync_copy(zbuf, acc_sh.at[pl.ds((cstart + t) * ZCH, ZCH)])

        plsc.subcore_barrier()

        start = cid * nblk_core + sid * per + jnp.minimum(sid, rem)
        cnt = per + (sid < rem).astype(jnp.int32)

        @pl.loop(0, cnt)
        def _(t):
            b = (start + t) * GBLK
            pltpu.sync_copy(dst_hbm.at[pl.ds(b, GBLK)], idx_v)
            pltpu.sync_copy(msg_hbm.at[pl.ds(b, GBLK)], val_v)
            pltpu.sync_copy(val_v, acc_sh.at[idx_v], add=True)

        plsc.subcore_barrier()

        @pl.loop(0, ccnt)
        def _(t):
            c16 = (cstart + t) * ZCH
            pltpu.sync_copy(acc_sh.at[pl.ds(c16, ZCH)],
                            out_hbm.at[cid, pl.ds(c16, ZCH)])

    return k(msg, dst)


def _p3_body(parts, x, g2, b2, out):
    aggr = parts[0] + parts[1]
    m = jnp.mean(aggr, axis=0)
    v = jnp.mean(aggr * aggr, axis=0) - m * m
    xn = g2[...] * (aggr - m[None]) * jax.lax.rsqrt(v[None] + 1e-5) + b2[...]
    out[...] = _silu(xn) + x[...]


def _tc_pass3(parts, x, g2, b2):
    return pl.pallas_call(
        _p3_body,
        out_shape=jax.ShapeDtypeStruct((N, D), jnp.float32),
    )(parts, x, g2, b2)


def kernel(x, edge_attr, edge_index, cart_dist, Wg1, bg1, Wg2, bg2,
           Wa1, ba1, Wa2, ba2, gamma1, beta1, gamma2, beta2):
    src = edge_index[0]
    dst = edge_index[1]

    xi, xj = _sc_gather(x, src, dst)

    wg1a, wg1b, wg1c = Wg1[:D], Wg1[D:2 * D], Wg1[2 * D:]
    wa1a, wa1b, wa1c = Wa1[:D], Wa1[D:2 * D], Wa1[2 * D:]
    bg1r, bg2r = bg1[None], bg2[None]
    ba1r, ba2r = ba1[None], ba2[None]
    g1r, b1r = gamma1[None], beta1[None]
    g2r, b2r = gamma2[None], beta2[None]

    stats = _tc_pass1(xi, xj, edge_attr, wg1a, wg1b, wg1c, bg1r, Wg2, bg2r)

    cd3 = cart_dist.reshape(E // BE, 1, BE)
    e_out, msg = _tc_pass2(xi, xj, edge_attr, cd3, stats,
                           wg1a, wg1b, wg1c, bg1r, Wg2, bg2r,
                           wa1a, wa1b, wa1c, ba1r, Wa2, ba2r, g1r, b1r)

    parts = _sc_scatter(msg, dst)
    x_out = _tc_pass3(parts, x, g2r, b2r)
    return (x_out, e_out)


# speedup vs baseline: 2.3963x; 1.0223x over previous
import functools

import jax
import jax.numpy as jnp
from jax import lax
from jax.experimental import pallas as pl
from jax.experimental.pallas import tpu as pltpu
from jax.experimental.pallas import tpu_sc as plsc

N = 10000
E = 320000
D = 128
RADIUS = 5.0

NC = 2
NS = 16
NW = NC * NS

GBLK = 128
BE = 1600


def _silu(z):
    return z * jax.nn.sigmoid(z)


def _sc_gather(xb, src, dst):
    nblk = E // GBLK
    per, rem = nblk // NW, nblk % NW
    mesh = plsc.VectorSubcoreMesh(core_axis_name="c", subcore_axis_name="s")

    @functools.partial(
        pl.kernel,
        out_type=(jax.ShapeDtypeStruct((E, D), jnp.float32),
                  jax.ShapeDtypeStruct((E, D), jnp.float32)),
        mesh=mesh,
        scratch_types=[
            pltpu.VMEM((GBLK,), jnp.int32),
            pltpu.VMEM((GBLK,), jnp.int32),
            pltpu.VMEM((GBLK, D), jnp.float32),
            pltpu.VMEM((GBLK, D), jnp.float32),
        ],
    )
    def k(x_hbm, src_hbm, dst_hbm, xi_hbm, xj_hbm, di_v, si_v, ri_v, rj_v):
        wid = lax.axis_index("s") * NC + lax.axis_index("c")
        start = wid * per + jnp.minimum(wid, rem)
        cnt = per + (wid < rem).astype(jnp.int32)

        @pl.loop(0, cnt)
        def _(t):
            b = (start + t) * GBLK
            pltpu.sync_copy(dst_hbm.at[pl.ds(b, GBLK)], di_v)
            pltpu.sync_copy(src_hbm.at[pl.ds(b, GBLK)], si_v)
            pltpu.sync_copy(x_hbm.at[di_v], ri_v)
            pltpu.sync_copy(x_hbm.at[si_v], rj_v)
            pltpu.sync_copy(ri_v, xi_hbm.at[pl.ds(b, GBLK)])
            pltpu.sync_copy(rj_v, xj_hbm.at[pl.ds(b, GBLK)])

    return k(xb, src, dst)


def _gate_u(xi, xj, ea, wg1a, wg1b, wg1c, bg1, wg2, bg2):
    pre = (jnp.dot(xi[...], wg1a[...], preferred_element_type=jnp.float32)
           + jnp.dot(xj[...], wg1b[...], preferred_element_type=jnp.float32)
           + jnp.dot(ea[...].astype(jnp.bfloat16), wg1c[...],
                     preferred_element_type=jnp.float32)
           + bg1[...])
    return (jnp.dot(_silu(pre).astype(jnp.bfloat16), wg2[...],
                    preferred_element_type=jnp.float32) + bg2[...])


def _p1_body(xi, xj, ea, wg1a, wg1b, wg1c, bg1, wg2, bg2, stats):
    i = pl.program_id(0)
    u = _gate_u(xi, xj, ea, wg1a, wg1b, wg1c, bg1, wg2, bg2)
    su = jnp.sum(u, axis=0)
    ss = jnp.sum(u * u, axis=0)
    blk = jnp.concatenate([su[None], ss[None], jnp.zeros((6, D), jnp.float32)], axis=0)

    @pl.when(i == 0)
    def _():
        stats[...] = jnp.zeros_like(stats)

    stats[...] += blk


def _tc_pass1(xi, xj, ea, wg1a, wg1b, wg1c, bg1, wg2, bg2):
    wspec = pl.BlockSpec((D, D), lambda i: (0, 0))
    bspec = pl.BlockSpec((1, D), lambda i: (0, 0))
    return pl.pallas_call(
        _p1_body,
        grid=(E // BE,),
        in_specs=[
            pl.BlockSpec((BE, D), lambda i: (i, 0)),
            pl.BlockSpec((BE, D), lambda i: (i, 0)),
            pl.BlockSpec((BE, D), lambda i: (i, 0)),
            wspec, wspec, wspec, bspec, wspec, bspec,
        ],
        out_specs=pl.BlockSpec((8, D), lambda i: (0, 0)),
        out_shape=jax.ShapeDtypeStruct((8, D), jnp.float32),
    )(xi, xj, ea, wg1a, wg1b, wg1c, bg1, wg2, bg2)


def _p2_body(xi, xj, ea, cd, stats, wg1a, wg1b, wg1c, bg1, wg2, bg2,
             wa1a, wa1b, wa1c, ba1, wa2, ba2, g1, b1, eo, msg):
    u = _gate_u(xi, xj, ea, wg1a, wg1b, wg1c, bg1, wg2, bg2)

    m = stats[0, :] * (1.0 / E)
    v = stats[1, :] * (1.0 / E) - m * m
    scale = g1[...] * jax.lax.rsqrt(v[None] + 1e-5)
    shift = b1[...] - m[None] * scale
    e_ij = jax.nn.sigmoid(u * scale + shift)

    d = cd[0, 0, :]
    env = 0.5 * (jnp.cos(d * (jnp.pi / RADIUS)) + 1.0) * (d < RADIUS).astype(jnp.float32)
    sigma = env[:, None] * e_ij

    sender = _gate_u(xi, xj, ea, wa1a, wa1b, wa1c, ba1, wa2, ba2)

    eo[...] = ea[...] + sigma
    msg[...] = sigma * sender


def _tc_pass2(xi, xj, ea, cd3, stats, wg1a, wg1b, wg1c, bg1, wg2, bg2,
              wa1a, wa1b, wa1c, ba1, wa2, ba2, g1, b1):
    wspec = pl.BlockSpec((D, D), lambda i: (0, 0))
    bspec = pl.BlockSpec((1, D), lambda i: (0, 0))
    espec = pl.BlockSpec((BE, D), lambda i: (i, 0))
    return pl.pallas_call(
        _p2_body,
        grid=(E // BE,),
        in_specs=[
            espec, espec, espec,
            pl.BlockSpec((1, 1, BE), lambda i: (i, 0, 0)),
            pl.BlockSpec((8, D), lambda i: (0, 0)),
            wspec, wspec, wspec, bspec, wspec, bspec,
            wspec, wspec, wspec, bspec, wspec, bspec,
            bspec, bspec,
        ],
        out_specs=[espec, espec],
        out_shape=[jax.ShapeDtypeStruct((E, D), jnp.float32),
                   jax.ShapeDtypeStruct((E, D), jnp.float32)],
    )(xi, xj, ea, cd3, stats, wg1a, wg1b, wg1c, bg1, wg2, bg2,
      wa1a, wa1b, wa1c, ba1, wa2, ba2, g1, b1)


ZCH = 16
NCHUNK = N // ZCH


def _sc_scatter(msg, dst):
    nblk_core = (E // GBLK) // NC
    per, rem = nblk_core // NS, nblk_core % NS
    cper, crem = NCHUNK // NS, NCHUNK % NS
    mesh = plsc.VectorSubcoreMesh(core_axis_name="c", subcore_axis_name="s")

    @functools.partial(
        pl.kernel,
        out_type=jax.ShapeDtypeStruct((NC, N, D), jnp.float32),
        mesh=mesh,
        scratch_types=[
            pltpu.VMEM((GBLK,), jnp.int32),
            pltpu.VMEM((GBLK, D), jnp.float32),
            pltpu.VMEM((ZCH, D), jnp.float32),
            pltpu.VMEM_SHARED((N, D), jnp.float32),
        ],
    )
    def k(msg_hbm, dst_hbm, out_hbm, idx_v, val_v, zbuf, acc_sh):
        cid = lax.axis_index("c")
        sid = lax.axis_index("s")

        for r in range(ZCH):
            for c in range(D // 16):
                zbuf[r, pl.ds(c * 16, 16)] = jnp.zeros((16,), jnp.float32)
        cstart = sid * cper + jnp.minimum(sid, crem)
        ccnt = cper + (sid < crem).astype(jnp.int32)

        @pl.loop(0, ccnt)
        def _(t):
            pltpu.sync_copy(zbuf, acc_sh.at[pl.ds((cstart + t) * ZCH, ZCH)])

        plsc.subcore_barrier()

        start = cid * nblk_core + sid * per + jnp.minimum(sid, rem)
        cnt = per + (sid < rem).astype(jnp.int32)

        @pl.loop(0, cnt)
        def _(t):
            b = (start + t) * GBLK
            pltpu.sync_copy(dst_hbm.at[pl.ds(b, GBLK)], idx_v)
            pltpu.sync_copy(msg_hbm.at[pl.ds(b, GBLK)], val_v)
            pltpu.sync_copy(val_v, acc_sh.at[idx_v], add=True)

        plsc.subcore_barrier()

        @pl.loop(0, ccnt)
        def _(t):
            c16 = (cstart + t) * ZCH
            pltpu.sync_copy(acc_sh.at[pl.ds(c16, ZCH)],
                            out_hbm.at[cid, pl.ds(c16, ZCH)])

    return k(msg, dst)


def _p3_body(parts, x, g2, b2, out):
    aggr = parts[0] + parts[1]
    m = jnp.mean(aggr, axis=0)
    v = jnp.mean(aggr * aggr, axis=0) - m * m
    xn = g2[...] * (aggr - m[None]) * jax.lax.rsqrt(v[None] + 1e-5) + b2[...]
    out[...] = _silu(xn) + x[...]


def _tc_pass3(parts, x, g2, b2):
    return pl.pallas_call(
        _p3_body,
        out_shape=jax.ShapeDtypeStruct((N, D), jnp.float32),
    )(parts, x, g2, b2)


def kernel(x, edge_attr, edge_index, cart_dist, Wg1, bg1, Wg2, bg2,
           Wa1, ba1, Wa2, ba2, gamma1, beta1, gamma2, beta2):
    src = edge_index[0]
    dst = edge_index[1]

    xi, xj = _sc_gather(x, src, dst)

    Wg1h, Wa1h = Wg1.astype(jnp.bfloat16), Wa1.astype(jnp.bfloat16)
    Wg2h, Wa2h = Wg2.astype(jnp.bfloat16), Wa2.astype(jnp.bfloat16)
    wg1a, wg1b, wg1c = Wg1h[:D], Wg1h[D:2 * D], Wg1h[2 * D:]
    wa1a, wa1b, wa1c = Wa1h[:D], Wa1h[D:2 * D], Wa1h[2 * D:]
    bg1r, bg2r = bg1[None], bg2[None]
    ba1r, ba2r = ba1[None], ba2[None]
    g1r, b1r = gamma1[None], beta1[None]
    g2r, b2r = gamma2[None], beta2[None]

    stats = _tc_pass1(xi, xj, edge_attr, wg1a, wg1b, wg1c, bg1r, Wg2h, bg2r)

    cd3 = cart_dist.reshape(E // BE, 1, BE)
    e_out, msg = _tc_pass2(xi, xj, edge_attr, cd3, stats,
                           wg1a, wg1b, wg1c, bg1r, Wg2h, bg2r,
                           wa1a, wa1b, wa1c, ba1r, Wa2h, ba2r, g1r, b1r)

    parts = _sc_scatter(msg, dst)
    x_out = _tc_pass3(parts, x, g2r, b2r)
    return (x_out, e_out)


# speedup vs baseline: 2.9089x; 1.2139x over previous
import functools

import jax
import jax.numpy as jnp
from jax import lax
from jax.experimental import pallas as pl
from jax.experimental.pallas import tpu as pltpu
from jax.experimental.pallas import tpu_sc as plsc

N = 10000
E = 320000
D = 128
RADIUS = 5.0

NC = 2
NS = 16
NW = NC * NS

GBLK = 80
EW = E // NW
NBW = EW // GBLK
BE = 1600


def _silu(z):
    return z * jax.nn.sigmoid(z)


def _sc_gather(xb, src, dst):
    mesh = plsc.VectorSubcoreMesh(core_axis_name="c", subcore_axis_name="s")

    @functools.partial(
        pl.kernel,
        out_type=(jax.ShapeDtypeStruct((E, D), jnp.float32),
                  jax.ShapeDtypeStruct((E, D), jnp.float32)),
        mesh=mesh,
        scratch_types=[
            pltpu.VMEM((2, GBLK), jnp.int32),
            pltpu.VMEM((2, GBLK), jnp.int32),
            pltpu.VMEM((2, GBLK, D), jnp.float32),
            pltpu.VMEM((2, GBLK, D), jnp.float32),
            pltpu.SemaphoreType.DMA,
            pltpu.SemaphoreType.DMA,
            pltpu.SemaphoreType.DMA,
            pltpu.SemaphoreType.DMA,
            pltpu.SemaphoreType.DMA,
        ],
    )
    def k(x_hbm, src_hbm, dst_hbm, xi_hbm, xj_hbm, di, si, ri, rj,
          sem_i0, sem_i1, sem_g, sem_w0, sem_w1):
        wid = lax.axis_index("s") * NC + lax.axis_index("c")
        base = wid * EW
        sems_i = (sem_i0, sem_i1)
        sems_w = (sem_w0, sem_w1)

        def idx_cp(t, b):
            off = base + t * GBLK
            return (pltpu.make_async_copy(dst_hbm.at[pl.ds(off, GBLK)],
                                          di.at[b], sems_i[b]),
                    pltpu.make_async_copy(src_hbm.at[pl.ds(off, GBLK)],
                                          si.at[b], sems_i[b]))

        def wb_cp(t, b):
            off = base + t * GBLK
            return (pltpu.make_async_copy(ri.at[b], xi_hbm.at[pl.ds(off, GBLK)],
                                          sems_w[b]),
                    pltpu.make_async_copy(rj.at[b], xj_hbm.at[pl.ds(off, GBLK)],
                                          sems_w[b]))

        for c in idx_cp(0, 0):
            c.start()

        @pl.loop(0, (NBW + 1) // 2)
        def _(it):
            for b in range(2):
                tt = it * 2 + b

                @pl.when(tt < NBW)
                def _():
                    for c in idx_cp(tt, b):
                        c.wait()

                    @pl.when(tt + 1 < NBW)
                    def _():
                        for c in idx_cp(tt + 1, 1 - b):
                            c.start()

                    @pl.when(tt >= 2)
                    def _():
                        for c in wb_cp(tt, b):
                            c.wait()

                    gi = pltpu.make_async_copy(x_hbm.at[di.at[b]], ri.at[b], sem_g)
                    gj = pltpu.make_async_copy(x_hbm.at[si.at[b]], rj.at[b], sem_g)
                    gi.start()
                    gj.start()
                    gi.wait()
                    gj.wait()

                    for c in wb_cp(tt, b):
                        c.start()

        for c in wb_cp(NBW - 2, (NBW - 2) % 2):
            c.wait()
        for c in wb_cp(NBW - 1, (NBW - 1) % 2):
            c.wait()

    return k(xb, src, dst)


def _gate_u(xi, xj, ea, wg1a, wg1b, wg1c, bg1, wg2, bg2):
    pre = (jnp.dot(xi[...], wg1a[...], preferred_element_type=jnp.float32)
           + jnp.dot(xj[...], wg1b[...], preferred_element_type=jnp.float32)
           + jnp.dot(ea[...].astype(jnp.bfloat16), wg1c[...],
                     preferred_element_type=jnp.float32)
           + bg1[...])
    return (jnp.dot(_silu(pre).astype(jnp.bfloat16), wg2[...],
                    preferred_element_type=jnp.float32) + bg2[...])


def _p1_body(xi, xj, ea, wg1a, wg1b, wg1c, bg1, wg2, bg2, stats):
    i = pl.program_id(0)
    u = _gate_u(xi, xj, ea, wg1a, wg1b, wg1c, bg1, wg2, bg2)
    su = jnp.sum(u, axis=0)
    ss = jnp.sum(u * u, axis=0)
    blk = jnp.concatenate([su[None], ss[None], jnp.zeros((6, D), jnp.float32)], axis=0)

    @pl.when(i == 0)
    def _():
        stats[...] = jnp.zeros_like(stats)

    stats[...] += blk


def _tc_pass1(xi, xj, ea, wg1a, wg1b, wg1c, bg1, wg2, bg2):
    wspec = pl.BlockSpec((D, D), lambda i: (0, 0))
    bspec = pl.BlockSpec((1, D), lambda i: (0, 0))
    return pl.pallas_call(
        _p1_body,
        grid=(E // BE,),
        in_specs=[
            pl.BlockSpec((BE, D), lambda i: (i, 0)),
            pl.BlockSpec((BE, D), lambda i: (i, 0)),
            pl.BlockSpec((BE, D), lambda i: (i, 0)),
            wspec, wspec, wspec, bspec, wspec, bspec,
        ],
        out_specs=pl.BlockSpec((8, D), lambda i: (0, 0)),
        out_shape=jax.ShapeDtypeStruct((8, D), jnp.float32),
    )(xi, xj, ea, wg1a, wg1b, wg1c, bg1, wg2, bg2)


def _p2_body(xi, xj, ea, cd, stats, wg1a, wg1b, wg1c, bg1, wg2, bg2,
             wa1a, wa1b, wa1c, ba1, wa2, ba2, g1, b1, eo, msg):
    u = _gate_u(xi, xj, ea, wg1a, wg1b, wg1c, bg1, wg2, bg2)

    m = stats[0, :] * (1.0 / E)
    v = stats[1, :] * (1.0 / E) - m * m
    scale = g1[...] * jax.lax.rsqrt(v[None] + 1e-5)
    shift = b1[...] - m[None] * scale
    e_ij = jax.nn.sigmoid(u * scale + shift)

    d = cd[0, 0, :]
    env = 0.5 * (jnp.cos(d * (jnp.pi / RADIUS)) + 1.0) * (d < RADIUS).astype(jnp.float32)
    sigma = env[:, None] * e_ij

    sender = _gate_u(xi, xj, ea, wa1a, wa1b, wa1c, ba1, wa2, ba2)

    eo[...] = ea[...] + sigma
    msg[...] = sigma * sender


def _tc_pass2(xi, xj, ea, cd3, stats, wg1a, wg1b, wg1c, bg1, wg2, bg2,
              wa1a, wa1b, wa1c, ba1, wa2, ba2, g1, b1):
    wspec = pl.BlockSpec((D, D), lambda i: (0, 0))
    bspec = pl.BlockSpec((1, D), lambda i: (0, 0))
    espec = pl.BlockSpec((BE, D), lambda i: (i, 0))
    return pl.pallas_call(
        _p2_body,
        grid=(E // BE,),
        in_specs=[
            espec, espec, espec,
            pl.BlockSpec((1, 1, BE), lambda i: (i, 0, 0)),
            pl.BlockSpec((8, D), lambda i: (0, 0)),
            wspec, wspec, wspec, bspec, wspec, bspec,
            wspec, wspec, wspec, bspec, wspec, bspec,
            bspec, bspec,
        ],
        out_specs=[espec, espec],
        out_shape=[jax.ShapeDtypeStruct((E, D), jnp.float32),
                   jax.ShapeDtypeStruct((E, D), jnp.float32)],
    )(xi, xj, ea, cd3, stats, wg1a, wg1b, wg1c, bg1, wg2, bg2,
      wa1a, wa1b, wa1c, ba1, wa2, ba2, g1, b1)


ZCH = 16
NROW = 624


def _sc_scatter(msg, dst):
    mesh = plsc.VectorSubcoreMesh(core_axis_name="c", subcore_axis_name="s")

    @functools.partial(
        pl.kernel,
        out_type=jax.ShapeDtypeStruct((NC, N, D), jnp.float32),
        mesh=mesh,
        scratch_types=[
            pltpu.VMEM((2, GBLK), jnp.int32),
            pltpu.VMEM((2, GBLK, D), jnp.float32),
            pltpu.VMEM((ZCH, D), jnp.float32),
            pltpu.VMEM_SHARED((N, D), jnp.float32),
            pltpu.SemaphoreType.DMA,
            pltpu.SemaphoreType.DMA,
            pltpu.SemaphoreType.DMA,
        ],
    )
    def k(msg_hbm, dst_hbm, out_hbm, di, mv, zbuf, acc_sh,
          sem_l0, sem_l1, sem_z):
        cid = lax.axis_index("c")
        sid = lax.axis_index("s")
        sems = (sem_l0, sem_l1)

        for r in range(ZCH):
            for c in range(D // 16):
                zbuf[r, pl.ds(c * 16, 16)] = jnp.zeros((16,), jnp.float32)
        nch = (NROW // ZCH) + (sid == 0).astype(jnp.int32)

        @pl.loop(0, nch)
        def _(t):
            roff = jnp.where(t < NROW // ZCH, sid * NROW + t * ZCH, NS * NROW)
            pltpu.make_async_copy(zbuf, acc_sh.at[pl.ds(roff, ZCH)],
                                  sem_z).start()

        @pl.loop(0, nch)
        def _(t):
            pltpu.make_async_copy(zbuf, acc_sh.at[pl.ds(0, ZCH)], sem_z).wait()

        plsc.subcore_barrier()

        base = (sid * NC + cid) * EW

        def ld(t, b):
            off = base + t * GBLK
            return (pltpu.make_async_copy(dst_hbm.at[pl.ds(off, GBLK)],
                                          di.at[b], sems[b]),
                    pltpu.make_async_copy(msg_hbm.at[pl.ds(off, GBLK)],
                                          mv.at[b], sems[b]))

        for c in ld(0, 0):
            c.start()

        @pl.loop(0, (NBW + 1) // 2)
        def _(it):
            for b in range(2):
                tt = it * 2 + b

                @pl.when(tt < NBW)
                def _():
                    for c in ld(tt, b):
                        c.wait()

                    @pl.when(tt + 1 < NBW)
                    def _():
                        for c in ld(tt + 1, 1 - b):
                            c.start()

                    pltpu.sync_copy(mv.at[b], acc_sh.at[di.at[b]], add=True)

        plsc.subcore_barrier()

        roff = sid * NROW
        wo = pltpu.make_async_copy(acc_sh.at[pl.ds(roff, NROW)],
                                   out_hbm.at[cid, pl.ds(roff, NROW)], sem_z)
        wo.start()

        @pl.when(sid == 0)
        def _():
            tl = pltpu.make_async_copy(acc_sh.at[pl.ds(NS * NROW, ZCH)],
                                       out_hbm.at[cid, pl.ds(NS * NROW, ZCH)],
                                       sem_z)
            tl.start()
            tl.wait()

        wo.wait()

    return k(msg, dst)


def _p3_body(parts, x, g2, b2, out):
    aggr = parts[0] + parts[1]
    m = jnp.mean(aggr, axis=0)
    v = jnp.mean(aggr * aggr, axis=0) - m * m
    xn = g2[...] * (aggr - m[None]) * jax.lax.rsqrt(v[None] + 1e-5) + b2[...]
    out[...] = _silu(xn) + x[...]


def _tc_pass3(parts, x, g2, b2):
    return pl.pallas_call(
        _p3_body,
        out_shape=jax.ShapeDtypeStruct((N, D), jnp.float32),
    )(parts, x, g2, b2)


def kernel(x, edge_attr, edge_index, cart_dist, Wg1, bg1, Wg2, bg2,
           Wa1, ba1, Wa2, ba2, gamma1, beta1, gamma2, beta2):
    src = edge_index[0]
    dst = edge_index[1]

    xi, xj = _sc_gather(x, src, dst)

    Wg1h, Wa1h = Wg1.astype(jnp.bfloat16), Wa1.astype(jnp.bfloat16)
    Wg2h, Wa2h = Wg2.astype(jnp.bfloat16), Wa2.astype(jnp.bfloat16)
    wg1a, wg1b, wg1c = Wg1h[:D], Wg1h[D:2 * D], Wg1h[2 * D:]
    wa1a, wa1b, wa1c = Wa1h[:D], Wa1h[D:2 * D], Wa1h[2 * D:]
    bg1r, bg2r = bg1[None], bg2[None]
    ba1r, ba2r = ba1[None], ba2[None]
    g1r, b1r = gamma1[None], beta1[None]
    g2r, b2r = gamma2[None], beta2[None]

    stats = _tc_pass1(xi, xj, edge_attr, wg1a, wg1b, wg1c, bg1r, Wg2h, bg2r)

    cd3 = cart_dist.reshape(E // BE, 1, BE)
    e_out, msg = _tc_pass2(xi, xj, edge_attr, cd3, stats,
                           wg1a, wg1b, wg1c, bg1r, Wg2h, bg2r,
                           wa1a, wa1b, wa1c, ba1r, Wa2h, ba2r, g1r, b1r)

    parts = _sc_scatter(msg, dst)
    x_out = _tc_pass3(parts, x, g2r, b2r)
    return (x_out, e_out)


# speedup vs baseline: 2.9520x; 1.0148x over previous
import functools

import jax
import jax.numpy as jnp
from jax import lax
from jax.experimental import pallas as pl
from jax.experimental.pallas import tpu as pltpu
from jax.experimental.pallas import tpu_sc as plsc

N = 10000
E = 320000
D = 128
RADIUS = 5.0

NC = 2
NS = 16
NW = NC * NS

EH = E // 2
GBLK = 40
EW = EH // NW
NBW = EW // GBLK
BE = 1600
HB = EH // BE


def _silu(z):
    return z * jax.nn.sigmoid(z)


def _sc_gather(xb, src, dst):
    mesh = plsc.VectorSubcoreMesh(core_axis_name="c", subcore_axis_name="s")

    @functools.partial(
        pl.kernel,
        out_type=(jax.ShapeDtypeStruct((EH, D), jnp.float32),
                  jax.ShapeDtypeStruct((EH, D), jnp.float32)),
        mesh=mesh,
        scratch_types=[
            pltpu.VMEM((2, GBLK), jnp.int32),
            pltpu.VMEM((2, GBLK), jnp.int32),
            pltpu.VMEM((2, GBLK, D), jnp.float32),
            pltpu.VMEM((2, GBLK, D), jnp.float32),
            pltpu.SemaphoreType.DMA,
            pltpu.SemaphoreType.DMA,
            pltpu.SemaphoreType.DMA,
            pltpu.SemaphoreType.DMA,
            pltpu.SemaphoreType.DMA,
        ],
    )
    def k(x_hbm, src_hbm, dst_hbm, xi_hbm, xj_hbm, di, si, ri, rj,
          sem_i0, sem_i1, sem_g, sem_w0, sem_w1):
        wid = lax.axis_index("s") * NC + lax.axis_index("c")
        base = wid * EW
        sems_i = (sem_i0, sem_i1)
        sems_w = (sem_w0, sem_w1)

        def idx_cp(t, b):
            off = base + t * GBLK
            return (pltpu.make_async_copy(dst_hbm.at[pl.ds(off, GBLK)],
                                          di.at[b], sems_i[b]),
                    pltpu.make_async_copy(src_hbm.at[pl.ds(off, GBLK)],
                                          si.at[b], sems_i[b]))

        def wb_cp(t, b):
            off = base + t * GBLK
            return (pltpu.make_async_copy(ri.at[b], xi_hbm.at[pl.ds(off, GBLK)],
                                          sems_w[b]),
                    pltpu.make_async_copy(rj.at[b], xj_hbm.at[pl.ds(off, GBLK)],
                                          sems_w[b]))

        for c in idx_cp(0, 0):
            c.start()

        @pl.loop(0, (NBW + 1) // 2)
        def _(it):
            for b in range(2):
                tt = it * 2 + b

                @pl.when(tt < NBW)
                def _():
                    for c in idx_cp(tt, b):
                        c.wait()

                    @pl.when(tt + 1 < NBW)
                    def _():
                        for c in idx_cp(tt + 1, 1 - b):
                            c.start()

                    @pl.when(tt >= 2)
                    def _():
                        for c in wb_cp(tt, b):
                            c.wait()

                    gi = pltpu.make_async_copy(x_hbm.at[di.at[b]], ri.at[b], sem_g)
                    gj = pltpu.make_async_copy(x_hbm.at[si.at[b]], rj.at[b], sem_g)
                    gi.start()
                    gj.start()
                    gi.wait()
                    gj.wait()

                    for c in wb_cp(tt, b):
                        c.start()

        for c in wb_cp(NBW - 2, (NBW - 2) % 2):
            c.wait()
        for c in wb_cp(NBW - 1, (NBW - 1) % 2):
            c.wait()

    return k(xb, src, dst)


def _gate_u(xi, xj, ea, wg1a, wg1b, wg1c, bg1, wg2, bg2):
    pre = (jnp.dot(xi[...], wg1a[...], preferred_element_type=jnp.float32)
           + jnp.dot(xj[...], wg1b[...], preferred_element_type=jnp.float32)
           + jnp.dot(ea[...].astype(jnp.bfloat16), wg1c[...],
                     preferred_element_type=jnp.float32)
           + bg1[...])
    return (jnp.dot(_silu(pre).astype(jnp.bfloat16), wg2[...],
                    preferred_element_type=jnp.float32) + bg2[...])


def _p1_body(xi, xj, ea, wg1a, wg1b, wg1c, bg1, wg2, bg2, stats):
    i = pl.program_id(0)
    u = _gate_u(xi, xj, ea, wg1a, wg1b, wg1c, bg1, wg2, bg2)
    su = jnp.sum(u, axis=0)
    ss = jnp.sum(u * u, axis=0)
    blk = jnp.concatenate([su[None], ss[None], jnp.zeros((6, D), jnp.float32)], axis=0)

    @pl.when(i == 0)
    def _():
        stats[...] = jnp.zeros_like(stats)

    stats[...] += blk


def _tc_pass1(xi, xj, ea, off, wg1a, wg1b, wg1c, bg1, wg2, bg2):
    wspec = pl.BlockSpec((D, D), lambda i: (0, 0))
    bspec = pl.BlockSpec((1, D), lambda i: (0, 0))
    hspec = pl.BlockSpec((BE, D), lambda i: (i, 0))
    return pl.pallas_call(
        _p1_body,
        grid=(HB,),
        in_specs=[
            hspec, hspec,
            pl.BlockSpec((BE, D), lambda i: (i + off, 0)),
            wspec, wspec, wspec, bspec, wspec, bspec,
        ],
        out_specs=pl.BlockSpec((8, D), lambda i: (0, 0)),
        out_shape=jax.ShapeDtypeStruct((8, D), jnp.float32),
    )(xi, xj, ea, wg1a, wg1b, wg1c, bg1, wg2, bg2)


def _p2_body(xi, xj, ea, cd, sA, sB, wg1a, wg1b, wg1c, bg1, wg2, bg2,
             wa1a, wa1b, wa1c, ba1, wa2, ba2, g1, b1, *rest):
    eo, msg = rest[-2], rest[-1]
    u = _gate_u(xi, xj, ea, wg1a, wg1b, wg1c, bg1, wg2, bg2)

    m = (sA[0, :] + sB[0, :]) * (1.0 / E)
    v = (sA[1, :] + sB[1, :]) * (1.0 / E) - m * m
    scale = g1[...] * jax.lax.rsqrt(v[None] + 1e-5)
    shift = b1[...] - m[None] * scale
    e_ij = jax.nn.sigmoid(u * scale + shift)

    d = cd[0, 0, :]
    env = 0.5 * (jnp.cos(d * (jnp.pi / RADIUS)) + 1.0) * (d < RADIUS).astype(jnp.float32)
    sigma = env[:, None] * e_ij

    sender = _gate_u(xi, xj, ea, wa1a, wa1b, wa1c, ba1, wa2, ba2)

    eo[...] = ea[...] + sigma
    msg[...] = sigma * sender


def _tc_pass2(xi, xj, ea, cd3, sA, sB, eo_in, off,
              wg1a, wg1b, wg1c, bg1, wg2, bg2,
              wa1a, wa1b, wa1c, ba1, wa2, ba2, g1, b1):
    wspec = pl.BlockSpec((D, D), lambda i: (0, 0))
    bspec = pl.BlockSpec((1, D), lambda i: (0, 0))
    hspec = pl.BlockSpec((BE, D), lambda i: (i, 0))
    ospec = pl.BlockSpec((BE, D), lambda i: (i + off, 0))
    sspec = pl.BlockSpec((8, D), lambda i: (0, 0))
    in_specs = [
        hspec, hspec, ospec,
        pl.BlockSpec((1, 1, BE), lambda i: (i + off, 0, 0)),
        sspec, sspec,
        wspec, wspec, wspec, bspec, wspec, bspec,
        wspec, wspec, wspec, bspec, wspec, bspec,
        bspec, bspec,
    ]
    args = [xi, xj, ea, cd3, sA, sB, wg1a, wg1b, wg1c, bg1, wg2, bg2,
            wa1a, wa1b, wa1c, ba1, wa2, ba2, g1, b1]
    aliases = {}
    if eo_in is not None:
        in_specs.append(ospec)
        args.append(eo_in)
        aliases = {20: 0}
    return pl.pallas_call(
        _p2_body,
        grid=(HB,),
        in_specs=in_specs,
        out_specs=[ospec, hspec],
        out_shape=[jax.ShapeDtypeStruct((E, D), jnp.float32),
                   jax.ShapeDtypeStruct((EH, D), jnp.float32)],
        input_output_aliases=aliases,
    )(*args)


ZCH = 16
NROW = 624


def _sc_scatter(msg, dst):
    mesh = plsc.VectorSubcoreMesh(core_axis_name="c", subcore_axis_name="s")

    @functools.partial(
        pl.kernel,
        out_type=jax.ShapeDtypeStruct((NC, N, D), jnp.float32),
        mesh=mesh,
        scratch_types=[
            pltpu.VMEM((2, GBLK), jnp.int32),
            pltpu.VMEM((2, GBLK, D), jnp.float32),
            pltpu.VMEM((ZCH, D), jnp.float32),
            pltpu.VMEM_SHARED((N, D), jnp.float32),
            pltpu.SemaphoreType.DMA,
            pltpu.SemaphoreType.DMA,
            pltpu.SemaphoreType.DMA,
        ],
    )
    def k(msg_hbm, dst_hbm, out_hbm, di, mv, zbuf, acc_sh,
          sem_l0, sem_l1, sem_z):
        cid = lax.axis_index("c")
        sid = lax.axis_index("s")
        sems = (sem_l0, sem_l1)

        for r in range(ZCH):
            for c in range(D // 16):
                zbuf[r, pl.ds(c * 16, 16)] = jnp.zeros((16,), jnp.float32)
        nch = (NROW // ZCH) + (sid == 0).astype(jnp.int32)

        @pl.loop(0, nch)
        def _(t):
            roff = jnp.where(t < NROW // ZCH, sid * NROW + t * ZCH, NS * NROW)
            pltpu.make_async_copy(zbuf, acc_sh.at[pl.ds(roff, ZCH)],
                                  sem_z).start()

        @pl.loop(0, nch)
        def _(t):
            pltpu.make_async_copy(zbuf, acc_sh.at[pl.ds(0, ZCH)], sem_z).wait()

        plsc.subcore_barrier()

        base = (sid * NC + cid) * EW

        def ld(t, b):
            off = base + t * GBLK
            return (pltpu.make_async_copy(dst_hbm.at[pl.ds(off, GBLK)],
                                          di.at[b], sems[b]),
                    pltpu.make_async_copy(msg_hbm.at[pl.ds(off, GBLK)],
                                          mv.at[b], sems[b]))

        for c in ld(0, 0):
            c.start()

        @pl.loop(0, (NBW + 1) // 2)
        def _(it):
            for b in range(2):
                tt = it * 2 + b

                @pl.when(tt < NBW)
                def _():
                    for c in ld(tt, b):
                        c.wait()

                    @pl.when(tt + 1 < NBW)
                    def _():
                        for c in ld(tt + 1, 1 - b):
                            c.start()

                    pltpu.sync_copy(mv.at[b], acc_sh.at[di.at[b]], add=True)

        plsc.subcore_barrier()

        roff = sid * NROW
        wo = pltpu.make_async_copy(acc_sh.at[pl.ds(roff, NROW)],
                                   out_hbm.at[cid, pl.ds(roff, NROW)], sem_z)
        wo.start()

        @pl.when(sid == 0)
        def _():
            tl = pltpu.make_async_copy(acc_sh.at[pl.ds(NS * NROW, ZCH)],
                                       out_hbm.at[cid, pl.ds(NS * NROW, ZCH)],
                                       sem_z)
            tl.start()
            tl.wait()

        wo.wait()

    return k(msg, dst)


def _p3_body(p0, p1, x, g2, b2, out):
    aggr = p0[0] + p0[1] + p1[0] + p1[1]
    m = jnp.mean(aggr, axis=0)
    v = jnp.mean(aggr * aggr, axis=0) - m * m
    xn = g2[...] * (aggr - m[None]) * jax.lax.rsqrt(v[None] + 1e-5) + b2[...]
    out[...] = _silu(xn) + x[...]


def _tc_pass3(p0, p1, x, g2, b2):
    return pl.pallas_call(
        _p3_body,
        out_shape=jax.ShapeDtypeStruct((N, D), jnp.float32),
    )(p0, p1, x, g2, b2)


def kernel(x, edge_attr, edge_index, cart_dist, Wg1, bg1, Wg2, bg2,
           Wa1, ba1, Wa2, ba2, gamma1, beta1, gamma2, beta2):
    src0, src1 = edge_index[0, :EH], edge_index[0, EH:]
    dst0, dst1 = edge_index[1, :EH], edge_index[1, EH:]

    xi0, xj0 = _sc_gather(x, src0, dst0)
    xi1, xj1 = _sc_gather(x, src1, dst1)

    Wg1h, Wa1h = Wg1.astype(jnp.bfloat16), Wa1.astype(jnp.bfloat16)
    Wg2h, Wa2h = Wg2.astype(jnp.bfloat16), Wa2.astype(jnp.bfloat16)
    wg1a, wg1b, wg1c = Wg1h[:D], Wg1h[D:2 * D], Wg1h[2 * D:]
    wa1a, wa1b, wa1c = Wa1h[:D], Wa1h[D:2 * D], Wa1h[2 * D:]
    bg1r, bg2r = bg1[None], bg2[None]
    ba1r, ba2r = ba1[None], ba2[None]
    g1r, b1r = gamma1[None], beta1[None]
    g2r, b2r = gamma2[None], beta2[None]

    sA = _tc_pass1(xi0, xj0, edge_attr, 0, wg1a, wg1b, wg1c, bg1r, Wg2h, bg2r)
    sB = _tc_pass1(xi1, xj1, edge_attr, HB, wg1a, wg1b, wg1c, bg1r, Wg2h, bg2r)

    cd3 = cart_dist.reshape(E // BE, 1, BE)
    eo0, msg0 = _tc_pass2(xi0, xj0, edge_attr, cd3, sA, sB, None, 0,
                          wg1a, wg1b, wg1c, bg1r, Wg2h, bg2r,
                          wa1a, wa1b, wa1c, ba1r, Wa2h, ba2r, g1r, b1r)
    p0 = _sc_scatter(msg0, dst0)
    e_out, msg1 = _tc_pass2(xi1, xj1, edge_attr, cd3, sA, sB, eo0, HB,
                            wg1a, wg1b, wg1c, bg1r, Wg2h, bg2r,
                            wa1a, wa1b, wa1c, ba1r, Wa2h, ba2r, g1r, b1r)
    p1 = _sc_scatter(msg1, dst1)

    x_out = _tc_pass3(p0, p1, x, g2r, b2r)
    return (x_out, e_out)


# speedup vs baseline: 3.3676x; 1.1408x over previous
import functools

import jax
import jax.numpy as jnp
from jax import lax
from jax.experimental import pallas as pl
from jax.experimental.pallas import tpu as pltpu
from jax.experimental.pallas import tpu_sc as plsc

N = 10000
E = 320000
D = 128
RADIUS = 5.0

NC = 2
NS = 16
NW = NC * NS

EH = E // 2
GBLK = 40
EW = EH // NW
NBW = EW // GBLK
BE = 1600
HB = EH // BE


def _silu(z):
    return z * jax.nn.sigmoid(z)


def _sc_gather(xb, src, dst):
    mesh = plsc.VectorSubcoreMesh(core_axis_name="c", subcore_axis_name="s")

    @functools.partial(
        pl.kernel,
        out_type=(jax.ShapeDtypeStruct((EH, D), jnp.float32),
                  jax.ShapeDtypeStruct((EH, D), jnp.float32)),
        mesh=mesh,
        scratch_types=[
            pltpu.VMEM((2, GBLK), jnp.int32),
            pltpu.VMEM((2, GBLK), jnp.int32),
            pltpu.VMEM((2, GBLK, D), jnp.float32),
            pltpu.VMEM((2, GBLK, D), jnp.float32),
            pltpu.VMEM_SHARED((N, D), jnp.float32),
            pltpu.SemaphoreType.DMA,
            pltpu.SemaphoreType.DMA,
            pltpu.SemaphoreType.DMA,
            pltpu.SemaphoreType.DMA,
            pltpu.SemaphoreType.DMA,
        ],
    )
    def k(x_hbm, src_hbm, dst_hbm, xi_hbm, xj_hbm, di, si, ri, rj, xsh,
          sem_i0, sem_i1, sem_g, sem_w0, sem_w1):
        wid = lax.axis_index("s") * NC + lax.axis_index("c")
        sid = lax.axis_index("s")
        base = wid * EW
        sems_i = (sem_i0, sem_i1)
        sems_w = (sem_w0, sem_w1)

        xcp = pltpu.make_async_copy(x_hbm.at[pl.ds(sid * NROW, NROW)],
                                    xsh.at[pl.ds(sid * NROW, NROW)], sem_g)
        xcp.start()

        @pl.when(sid == 0)
        def _():
            tl = pltpu.make_async_copy(x_hbm.at[pl.ds(NS * NROW, ZCH)],
                                       xsh.at[pl.ds(NS * NROW, ZCH)], sem_g)
            tl.start()
            tl.wait()

        xcp.wait()
        plsc.subcore_barrier()

        def idx_cp(t, b):
            off = base + t * GBLK
            return (pltpu.make_async_copy(dst_hbm.at[pl.ds(off, GBLK)],
                                          di.at[b], sems_i[b]),
                    pltpu.make_async_copy(src_hbm.at[pl.ds(off, GBLK)],
                                          si.at[b], sems_i[b]))

        def wb_cp(t, b):
            off = base + t * GBLK
            return (pltpu.make_async_copy(ri.at[b], xi_hbm.at[pl.ds(off, GBLK)],
                                          sems_w[b]),
                    pltpu.make_async_copy(rj.at[b], xj_hbm.at[pl.ds(off, GBLK)],
                                          sems_w[b]))

        for c in idx_cp(0, 0):
            c.start()

        @pl.loop(0, (NBW + 1) // 2)
        def _(it):
            for b in range(2):
                tt = it * 2 + b

                @pl.when(tt < NBW)
                def _():
                    for c in idx_cp(tt, b):
                        c.wait()

                    @pl.when(tt + 1 < NBW)
                    def _():
                        for c in idx_cp(tt + 1, 1 - b):
                            c.start()

                    @pl.when(tt >= 2)
                    def _():
                        for c in wb_cp(tt, b):
                            c.wait()

                    pltpu.sync_copy(xsh.at[di.at[b]], ri.at[b])
                    pltpu.sync_copy(xsh.at[si.at[b]], rj.at[b])

                    for c in wb_cp(tt, b):
                        c.start()

        for c in wb_cp(NBW - 2, (NBW - 2) % 2):
            c.wait()
        for c in wb_cp(NBW - 1, (NBW - 1) % 2):
            c.wait()

    return k(xb, src, dst)


def _gate_u(xi, xj, ea, wg1a, wg1b, wg1c, bg1, wg2, bg2):
    pre = (jnp.dot(xi[...], wg1a[...], preferred_element_type=jnp.float32)
           + jnp.dot(xj[...], wg1b[...], preferred_element_type=jnp.float32)
           + jnp.dot(ea[...].astype(jnp.bfloat16), wg1c[...],
                     preferred_element_type=jnp.float32)
           + bg1[...])
    return (jnp.dot(_silu(pre).astype(jnp.bfloat16), wg2[...],
                    preferred_element_type=jnp.float32) + bg2[...])


def _p1_body(xi, xj, ea, wg1a, wg1b, wg1c, bg1, wg2, bg2, stats):
    i = pl.program_id(0)
    u = _gate_u(xi, xj, ea, wg1a, wg1b, wg1c, bg1, wg2, bg2)
    su = jnp.sum(u, axis=0)
    ss = jnp.sum(u * u, axis=0)
    blk = jnp.concatenate([su[None], ss[None], jnp.zeros((6, D), jnp.float32)], axis=0)

    @pl.when(i == 0)
    def _():
        stats[...] = jnp.zeros_like(stats)

    stats[...] += blk


def _tc_pass1(xi, xj, ea, off, wg1a, wg1b, wg1c, bg1, wg2, bg2):
    wspec = pl.BlockSpec((D, D), lambda i: (0, 0))
    bspec = pl.BlockSpec((1, D), lambda i: (0, 0))
    hspec = pl.BlockSpec((BE, D), lambda i: (i, 0))
    return pl.pallas_call(
        _p1_body,
        grid=(HB,),
        in_specs=[
            hspec, hspec,
            pl.BlockSpec((BE, D), lambda i: (i + off, 0)),
            wspec, wspec, wspec, bspec, wspec, bspec,
        ],
        out_specs=pl.BlockSpec((8, D), lambda i: (0, 0)),
        out_shape=jax.ShapeDtypeStruct((8, D), jnp.float32),
    )(xi, xj, ea, wg1a, wg1b, wg1c, bg1, wg2, bg2)


def _p2_body(xi, xj, ea, cd, sA, sB, wg1a, wg1b, wg1c, bg1, wg2, bg2,
             wa1a, wa1b, wa1c, ba1, wa2, ba2, g1, b1, *rest):
    eo, msg = rest[-2], rest[-1]
    u = _gate_u(xi, xj, ea, wg1a, wg1b, wg1c, bg1, wg2, bg2)

    m = (sA[0, :] + sB[0, :]) * (1.0 / E)
    v = (sA[1, :] + sB[1, :]) * (1.0 / E) - m * m
    scale = g1[...] * jax.lax.rsqrt(v[None] + 1e-5)
    shift = b1[...] - m[None] * scale
    e_ij = jax.nn.sigmoid(u * scale + shift)

    d = cd[0, 0, :]
    env = 0.5 * (jnp.cos(d * (jnp.pi / RADIUS)) + 1.0) * (d < RADIUS).astype(jnp.float32)
    sigma = env[:, None] * e_ij

    sender = _gate_u(xi, xj, ea, wa1a, wa1b, wa1c, ba1, wa2, ba2)

    eo[...] = ea[...] + sigma
    msg[...] = sigma * sender


def _tc_pass2(xi, xj, ea, cd3, sA, sB, eo_in, off,
              wg1a, wg1b, wg1c, bg1, wg2, bg2,
              wa1a, wa1b, wa1c, ba1, wa2, ba2, g1, b1):
    wspec = pl.BlockSpec((D, D), lambda i: (0, 0))
    bspec = pl.BlockSpec((1, D), lambda i: (0, 0))
    hspec = pl.BlockSpec((BE, D), lambda i: (i, 0))
    ospec = pl.BlockSpec((BE, D), lambda i: (i + off, 0))
    sspec = pl.BlockSpec((8, D), lambda i: (0, 0))
    in_specs = [
        hspec, hspec, ospec,
        pl.BlockSpec((1, 1, BE), lambda i: (i + off, 0, 0)),
        sspec, sspec,
        wspec, wspec, wspec, bspec, wspec, bspec,
        wspec, wspec, wspec, bspec, wspec, bspec,
        bspec, bspec,
    ]
    args = [xi, xj, ea, cd3, sA, sB, wg1a, wg1b, wg1c, bg1, wg2, bg2,
            wa1a, wa1b, wa1c, ba1, wa2, ba2, g1, b1]
    aliases = {}
    if eo_in is not None:
        in_specs.append(ospec)
        args.append(eo_in)
        aliases = {20: 0}
    return pl.pallas_call(
        _p2_body,
        grid=(HB,),
        in_specs=in_specs,
        out_specs=[ospec, hspec],
        out_shape=[jax.ShapeDtypeStruct((E, D), jnp.float32),
                   jax.ShapeDtypeStruct((EH, D), jnp.float32)],
        input_output_aliases=aliases,
    )(*args)


ZCH = 16
NROW = 624


def _sc_scatter(msg, dst):
    mesh = plsc.VectorSubcoreMesh(core_axis_name="c", subcore_axis_name="s")

    @functools.partial(
        pl.kernel,
        out_type=jax.ShapeDtypeStruct((NC, N, D), jnp.float32),
        mesh=mesh,
        scratch_types=[
            pltpu.VMEM((2, GBLK), jnp.int32),
            pltpu.VMEM((2, GBLK, D), jnp.float32),
            pltpu.VMEM((ZCH, D), jnp.float32),
            pltpu.VMEM_SHARED((N, D), jnp.float32),
            pltpu.SemaphoreType.DMA,
            pltpu.SemaphoreType.DMA,
            pltpu.SemaphoreType.DMA,
        ],
    )
    def k(msg_hbm, dst_hbm, out_hbm, di, mv, zbuf, acc_sh,
          sem_l0, sem_l1, sem_z):
        cid = lax.axis_index("c")
        sid = lax.axis_index("s")
        sems = (sem_l0, sem_l1)

        for r in range(ZCH):
            for c in range(D // 16):
                zbuf[r, pl.ds(c * 16, 16)] = jnp.zeros((16,), jnp.float32)
        nch = (NROW // ZCH) + (sid == 0).astype(jnp.int32)

        @pl.loop(0, nch)
        def _(t):
            roff = jnp.where(t < NROW // ZCH, sid * NROW + t * ZCH, NS * NROW)
            pltpu.make_async_copy(zbuf, acc_sh.at[pl.ds(roff, ZCH)],
                                  sem_z).start()

        @pl.loop(0, nch)
        def _(t):
            pltpu.make_async_copy(zbuf, acc_sh.at[pl.ds(0, ZCH)], sem_z).wait()

        plsc.subcore_barrier()

        base = (sid * NC + cid) * EW

        def ld(t, b):
            off = base + t * GBLK
            return (pltpu.make_async_copy(dst_hbm.at[pl.ds(off, GBLK)],
                                          di.at[b], sems[b]),
                    pltpu.make_async_copy(msg_hbm.at[pl.ds(off, GBLK)],
                                          mv.at[b], sems[b]))

        for c in ld(0, 0):
            c.start()

        @pl.loop(0, (NBW + 1) // 2)
        def _(it):
            for b in range(2):
                tt = it * 2 + b

                @pl.when(tt < NBW)
                def _():
                    for c in ld(tt, b):
                        c.wait()

                    @pl.when(tt + 1 < NBW)
                    def _():
                        for c in ld(tt + 1, 1 - b):
                            c.start()

                    pltpu.sync_copy(mv.at[b], acc_sh.at[di.at[b]], add=True)

        plsc.subcore_barrier()

        roff = sid * NROW
        wo = pltpu.make_async_copy(acc_sh.at[pl.ds(roff, NROW)],
                                   out_hbm.at[cid, pl.ds(roff, NROW)], sem_z)
        wo.start()

        @pl.when(sid == 0)
        def _():
            tl = pltpu.make_async_copy(acc_sh.at[pl.ds(NS * NROW, ZCH)],
                                       out_hbm.at[cid, pl.ds(NS * NROW, ZCH)],
                                       sem_z)
            tl.start()
            tl.wait()

        wo.wait()

    return k(msg, dst)


def _p3_body(p0, p1, x, g2, b2, out):
    aggr = p0[0] + p0[1] + p1[0] + p1[1]
    m = jnp.mean(aggr, axis=0)
    v = jnp.mean(aggr * aggr, axis=0) - m * m
    xn = g2[...] * (aggr - m[None]) * jax.lax.rsqrt(v[None] + 1e-5) + b2[...]
    out[...] = _silu(xn) + x[...]


def _tc_pass3(p0, p1, x, g2, b2):
    return pl.pallas_call(
        _p3_body,
        out_shape=jax.ShapeDtypeStruct((N, D), jnp.float32),
    )(p0, p1, x, g2, b2)


def kernel(x, edge_attr, edge_index, cart_dist, Wg1, bg1, Wg2, bg2,
           Wa1, ba1, Wa2, ba2, gamma1, beta1, gamma2, beta2):
    src0, src1 = edge_index[0, :EH], edge_index[0, EH:]
    dst0, dst1 = edge_index[1, :EH], edge_index[1, EH:]

    xi0, xj0 = _sc_gather(x, src0, dst0)
    xi1, xj1 = _sc_gather(x, src1, dst1)

    Wg1h, Wa1h = Wg1.astype(jnp.bfloat16), Wa1.astype(jnp.bfloat16)
    Wg2h, Wa2h = Wg2.astype(jnp.bfloat16), Wa2.astype(jnp.bfloat16)
    wg1a, wg1b, wg1c = Wg1h[:D], Wg1h[D:2 * D], Wg1h[2 * D:]
    wa1a, wa1b, wa1c = Wa1h[:D], Wa1h[D:2 * D], Wa1h[2 * D:]
    bg1r, bg2r = bg1[None], bg2[None]
    ba1r, ba2r = ba1[None], ba2[None]
    g1r, b1r = gamma1[None], beta1[None]
    g2r, b2r = gamma2[None], beta2[None]

    sA = _tc_pass1(xi0, xj0, edge_attr, 0, wg1a, wg1b, wg1c, bg1r, Wg2h, bg2r)
    sB = _tc_pass1(xi1, xj1, edge_attr, HB, wg1a, wg1b, wg1c, bg1r, Wg2h, bg2r)

    cd3 = cart_dist.reshape(E // BE, 1, BE)
    eo0, msg0 = _tc_pass2(xi0, xj0, edge_attr, cd3, sA, sB, None, 0,
                          wg1a, wg1b, wg1c, bg1r, Wg2h, bg2r,
                          wa1a, wa1b, wa1c, ba1r, Wa2h, ba2r, g1r, b1r)
    p0 = _sc_scatter(msg0, dst0)
    e_out, msg1 = _tc_pass2(xi1, xj1, edge_attr, cd3, sA, sB, eo0, HB,
                            wg1a, wg1b, wg1c, bg1r, Wg2h, bg2r,
                            wa1a, wa1b, wa1c, ba1r, Wa2h, ba2r, g1r, b1r)
    p1 = _sc_scatter(msg1, dst1)

    x_out = _tc_pass3(p0, p1, x, g2r, b2r)
    return (x_out, e_out)
